# baseline (speedup 1.0000x reference)
import functools

import jax
import jax.numpy as jnp
from jax import lax
from jax.experimental import pallas as pl
from jax.experimental.pallas import tpu as pltpu
from jax.experimental.pallas import tpu_sc as plsc

N = 32768
D = 768
E = 8
LOSS_COEF = 0.01
GATE_EPS = 1e-10

BN = 8192
NW = 32
TPW = N // NW
NGROUPS = TPW // 16


def _matmul_kernel(x_ref, wt_ref, bt_ref, geot_ref):
    geot_ref[:, :] = lax.dot_general(
        wt_ref[:, :], x_ref[:, :],
        dimension_numbers=(((1,), (1,)), ((), ())),
        preferred_element_type=jnp.float32,
    ) + bt_ref[:, :]


@jax.jit
def _matmul(x, WcatT, bcatT):
    return pl.pallas_call(
        _matmul_kernel,
        grid=(N // BN,),
        in_specs=[
            pl.BlockSpec((BN, D), lambda i: (i, 0)),
            pl.BlockSpec((2 * E, D), lambda i: (0, 0)),
            pl.BlockSpec((2 * E, 1), lambda i: (0, 0)),
        ],
        out_specs=pl.BlockSpec((2 * E, BN), lambda i: (0, i)),
        out_shape=jax.ShapeDtypeStruct((2 * E, N), jnp.float32),
    )(x, WcatT, bcatT)


def _routing_kernel(geot_hbm, out_hbm, gst_hbm, imp_hbm, load_hbm,
                    geot_v, pst_v, out_v, stat_v):
    wid = lax.axis_index("s") * 2 + lax.axis_index("c")

    pltpu.sync_copy(geot_hbm.at[:, pl.ds(wid * TPW, TPW)], geot_v)

    zero16 = jnp.zeros((16,), jnp.float32)
    neg_inf = jnp.full((16,), -jnp.inf, jnp.float32)

    def group(t, carry):
        base = t * 16
        g = [geot_v[e, pl.ds(base, 16)] for e in range(E)]
        eo = [geot_v[E + e, pl.ds(base, 16)] for e in range(E)]

        m1 = g[0]
        for e in range(1, E):
            m1 = jnp.maximum(m1, g[e])
        a1 = jnp.full((16,), E, jnp.int32)
        for e in range(E - 1, -1, -1):
            a1 = jnp.where(g[e] == m1, jnp.full((16,), e, jnp.int32), a1)
        g2 = [jnp.where(a1 == e, neg_inf, g[e]) for e in range(E)]
        m2 = g2[0]
        for e in range(1, E):
            m2 = jnp.maximum(m2, g2[e])
        a2 = jnp.full((16,), E, jnp.int32)
        for e in range(E - 1, -1, -1):
            a2 = jnp.where(g2[e] == m2, jnp.full((16,), e, jnp.int32), a2)

        mx = jnp.maximum(m1, zero16)
        ex = []
        s = zero16
        for e in range(E):
            keep = (a1 == e) | (a2 == e)
            me = jnp.where(keep, g[e], zero16)
            x_e = jnp.exp(me - mx)
            ex.append(x_e)
            s = s + x_e
        r = 1.0 / s

        acc = zero16
        new_carry = []
        for e in range(E):
            p_e = ex[e] * r
            pst_v[e, pl.ds(base, 16)] = p_e
            acc = acc + p_e * eo[e]
            new_carry.append(carry[e] + p_e)
        for e in range(E):
            p_e = ex[e] * r
            new_carry.append(
                carry[E + e] + jnp.where(p_e > 0, 1.0, 0.0).astype(jnp.float32))
        out_v[pl.ds(base, 16)] = acc
        return tuple(new_carry)

    init = tuple(jnp.zeros((16,), jnp.float32) for _ in range(2 * E))
    stats = lax.fori_loop(0, NGROUPS, group, init)

    for e in range(E):
        stat_v[pl.ds(e * 16, 16)] = stats[e]
        stat_v[pl.ds(128 + e * 16, 16)] = stats[E + e]

    pltpu.sync_copy(out_v, out_hbm.at[pl.ds(wid * TPW, TPW)])
    pltpu.sync_copy(pst_v, gst_hbm.at[:, pl.ds(wid * TPW, TPW)])
    pltpu.sync_copy(stat_v.at[pl.ds(0, 128)], imp_hbm.at[wid])
    pltpu.sync_copy(stat_v.at[pl.ds(128, 128)], load_hbm.at[wid])


@jax.jit
def _routing(geot):
    f = functools.partial(
        pl.kernel,
        out_type=[
            jax.ShapeDtypeStruct((N,), jnp.float32),
            jax.ShapeDtypeStruct((E, N), jnp.float32),
            jax.ShapeDtypeStruct((NW, 128), jnp.float32),
            jax.ShapeDtypeStruct((NW, 128), jnp.float32),
        ],
        mesh=plsc.VectorSubcoreMesh(core_axis_name="c", subcore_axis_name="s"),
        scratch_types=[
            pltpu.VMEM((2 * E, TPW), jnp.float32),
            pltpu.VMEM((E, TPW), jnp.float32),
            pltpu.VMEM((TPW,), jnp.float32),
            pltpu.VMEM((2 * 128,), jnp.float32),
        ],
    )(_routing_kernel)
    return f(geot)


def _finalize_kernel(gst_ref, imp_ref, load_ref, gs_ref, loss_ref):
    i = pl.program_id(0)
    gs_ref[:, :] = gst_ref[:, :].T

    @pl.when(i == 0)
    def _():
        i0 = lax.broadcasted_iota(jnp.int32, (128, E), 0)
        i1 = lax.broadcasted_iota(jnp.int32, (128, E), 1)
        seg = (i0 // 16 == i1).astype(jnp.float32)

        def cv2(part_ref):
            a = jnp.sum(part_ref[:, :], axis=0, keepdims=True)
            v = jnp.dot(a, seg, preferred_element_type=jnp.float32)
            mean = jnp.sum(v) / E
            var = jnp.sum((v - mean) ** 2) / (E - 1)
            return var / (mean * mean + GATE_EPS)

        loss = (cv2(imp_ref) + cv2(load_ref)) * LOSS_COEF
        loss_ref[:, :] = jnp.full((1, 1), loss, dtype=jnp.float32)


@jax.jit
def _finalize(gst, impP, loadP):
    return pl.pallas_call(
        _finalize_kernel,
        grid=(N // BN,),
        in_specs=[
            pl.BlockSpec((E, BN), lambda i: (0, i)),
            pl.BlockSpec((NW, 128), lambda i: (0, 0)),
            pl.BlockSpec((NW, 128), lambda i: (0, 0)),
        ],
        out_specs=[
            pl.BlockSpec((BN, E), lambda i: (i, 0)),
            pl.BlockSpec((1, 1), lambda i: (0, 0)),
        ],
        out_shape=[
            jax.ShapeDtypeStruct((N, E), jnp.float32),
            jax.ShapeDtypeStruct((1, 1), jnp.float32),
        ],
    )(gst, impP, loadP)


def kernel(x, W_gate, b_gate, W_experts, b_experts):
    WcatT = jnp.concatenate([W_gate, W_experts], axis=1).T
    bcatT = jnp.concatenate([b_gate, b_experts]).reshape(2 * E, 1)
    geot = _matmul(x, WcatT, bcatT)
    out, gst, impP, loadP = _routing(geot)
    gs, loss = _finalize(gst, impP, loadP)
    return out, loss[0, 0], gs
